# Initial kernel scaffold; baseline (speedup 1.0000x reference)
import jax
import jax.numpy as jnp
from jax import lax
from jax.experimental import pallas as pl
from jax.experimental.pallas import tpu as pltpu
from jax.experimental.pallas import tpu_sc as plsc

N = 10000
E = 320000
HEADS = 4
OUT_DIM = 32
HD = HEADS * OUT_DIM
TW = 144

NC = 2
NS = 16
NW = NC * NS
CHUNK = 128
CHUNKS_PER_TILE = 79
E_PAD = NW * CHUNKS_PER_TILE * CHUNK
ROWS_PER_TILE = N // NS
BN = 2000


def _tc_prep_kernel(x_ref, wl_ref, wr_ref, br_ref, bias_ref, wai_ref, waj_ref,
                    x2_ref, ai_ref, res_ref):
    x = x_ref[...]
    xl = jnp.dot(x, wl_ref[...].T, preferred_element_type=jnp.float32)
    x2_ref[:, 0:HD] = xl
    x2_ref[:, HD:TW] = jnp.dot(xl, waj_ref[...],
                               preferred_element_type=jnp.float32)
    ai_ref[...] = jnp.dot(xl, wai_ref[...], preferred_element_type=jnp.float32)
    res_ref[...] = (jnp.dot(x, wr_ref[...].T,
                            preferred_element_type=jnp.float32)
                    + br_ref[...] + bias_ref[...])


def _tc_combine_kernel(acc_ref, x2_ref, ai_ref, res_ref, out_ref):
    rows = lax.broadcasted_iota(jnp.int32, (16, HD), 0)
    cols = lax.broadcasted_iota(jnp.int32, (16, HD), 1)
    expand = (rows == cols // OUT_DIM).astype(jnp.float32)

    acc = acc_ref[0] + acc_ref[1]
    xl = x2_ref[:, 0:HD]
    s_self = ai_ref[...] + x2_ref[:, HD:TW]
    den_self = jnp.exp(jnp.maximum(s_self, 0.2 * s_self))
    num = acc[:, 0:HD] + jnp.dot(den_self, expand,
                                 preferred_element_type=jnp.float32) * xl
    den = jnp.dot(acc[:, HD:TW] + den_self, expand,
                  preferred_element_type=jnp.float32) + 1e-16
    out_ref[...] = num / den + res_ref[...]


def _bcast_lane(vec, lane):
    idx = jnp.full((16, 1), lane, dtype=jnp.int32)
    return lax.gather(
        vec, idx,
        lax.GatherDimensionNumbers(offset_dims=(), collapsed_slice_dims=(0,),
                                   start_index_map=(0,)),
        (1,), mode=lax.GatherScatterMode.PROMISE_IN_BOUNDS)


def _sc_edge_kernel(row_hbm, col_hbm, ai_hbm, x2_hbm, z_hbm, out_hbm,
                    ridx, cidx, aig, x2g, acc, sem_a, sem_b):
    cid = lax.axis_index("c")
    sid = lax.axis_index("s")
    wid = sid * NC + cid

    pltpu.sync_copy(z_hbm.at[pl.ds(sid * ROWS_PER_TILE, ROWS_PER_TILE)],
                    acc.at[pl.ds(sid * ROWS_PER_TILE, ROWS_PER_TILE)])
    plsc.subcore_barrier()

    chunk0 = wid * CHUNKS_PER_TILE

    @pl.loop(0, CHUNKS_PER_TILE)
    def _chunk(g):
        gi = chunk0 + g
        pltpu.sync_copy(row_hbm.at[pl.ds(gi, 1)], ridx)
        pltpu.sync_copy(col_hbm.at[pl.ds(gi, 1)], cidx)
        c1 = pltpu.async_copy(ai_hbm.at[ridx.at[0]], aig, sem_a)
        c2 = pltpu.async_copy(x2_hbm.at[cidx.at[0]], x2g, sem_b)
        c1.wait()
        c2.wait()

        @pl.loop(0, CHUNK // 16)
        def _grp(grp):
            ri = ridx[pl.ds(0, 1), pl.ds(grp * 16, 16)].reshape((16,))
            ci = cidx[pl.ds(0, 1), pl.ds(grp * 16, 16)].reshape((16,))
            vmask = jnp.where(ri != ci, 1.0, 0.0).astype(jnp.float32)
            for j in range(16):
                e = grp * 16 + j
                ai = aig[pl.ds(e, 1), :].reshape((16,))
                aj = x2g[pl.ds(e, 1), pl.ds(HD, 16)].reshape((16,))
                s = ai + aj
                s = jnp.maximum(s, 0.2 * s)
                esc = jnp.exp(s) * _bcast_lane(vmask, j)
                x2g[pl.ds(e, 1), pl.ds(HD, 16)] = esc.reshape((1, 16))
                for h in range(HEADS):
                    sc_h = _bcast_lane(esc, h)
                    for c in (2 * h, 2 * h + 1):
                        sl = (pl.ds(e, 1), pl.ds(c * 16, 16))
                        v = x2g[sl].reshape((16,))
                        x2g[sl] = (v * sc_h).reshape((1, 16))

        pltpu.sync_copy(x2g, acc.at[ridx.at[0]], add=True)

    plsc.subcore_barrier()
    pltpu.sync_copy(acc.at[pl.ds(sid * ROWS_PER_TILE, ROWS_PER_TILE)],
                    out_hbm.at[cid].at[pl.ds(sid * ROWS_PER_TILE,
                                             ROWS_PER_TILE)])


def kernel(x, W_lin, att_i, att_j, bias, W_res, b_res, edge_index):
    f32 = jnp.float32

    att_i_f = att_i.reshape(HEADS, OUT_DIM).astype(f32)
    att_j_f = att_j.reshape(HEADS, OUT_DIM).astype(f32)
    wai = jnp.zeros((HD, 16), f32)
    waj = jnp.zeros((HD, 16), f32)
    for h in range(HEADS):
        wai = wai.at[h * OUT_DIM:(h + 1) * OUT_DIM, h].set(att_i_f[h])
        waj = waj.at[h * OUT_DIM:(h + 1) * OUT_DIM, h].set(att_j_f[h])

    pad = jnp.zeros((E_PAD - E,), jnp.int32)
    row2d = jnp.concatenate([edge_index[0], pad]).reshape(-1, CHUNK)
    col2d = jnp.concatenate([edge_index[1], pad]).reshape(-1, CHUNK)
    zeros_tab = jnp.zeros((N, TW), f32)

    grid = (N // BN,)
    x2, ai_tab, res = pl.pallas_call(
        _tc_prep_kernel,
        grid=grid,
        in_specs=[
            pl.BlockSpec((BN, HD), lambda i: (i, 0)),
            pl.BlockSpec((HD, HD), lambda i: (0, 0)),
            pl.BlockSpec((HD, HD), lambda i: (0, 0)),
            pl.BlockSpec((HD,), lambda i: (0,)),
            pl.BlockSpec((HD,), lambda i: (0,)),
            pl.BlockSpec((HD, 16), lambda i: (0, 0)),
            pl.BlockSpec((HD, 16), lambda i: (0, 0)),
        ],
        out_specs=[
            pl.BlockSpec((BN, TW), lambda i: (i, 0)),
            pl.BlockSpec((BN, 16), lambda i: (i, 0)),
            pl.BlockSpec((BN, HD), lambda i: (i, 0)),
        ],
        out_shape=[
            jax.ShapeDtypeStruct((N, TW), f32),
            jax.ShapeDtypeStruct((N, 16), f32),
            jax.ShapeDtypeStruct((N, HD), f32),
        ],
    )(x, W_lin, W_res, b_res, bias, wai, waj)

    mesh = plsc.VectorSubcoreMesh(core_axis_name="c", subcore_axis_name="s")
    sc = pl.kernel(
        _sc_edge_kernel,
        out_type=jax.ShapeDtypeStruct((NC, N, TW), f32),
        mesh=mesh,
        scratch_types=[
            pltpu.VMEM((1, CHUNK), jnp.int32),
            pltpu.VMEM((1, CHUNK), jnp.int32),
            pltpu.VMEM((CHUNK, 16), f32),
            pltpu.VMEM((CHUNK, TW), f32),
            pltpu.VMEM_SHARED((N, TW), f32),
            pltpu.SemaphoreType.DMA,
            pltpu.SemaphoreType.DMA,
        ],
    )
    acc = sc(row2d, col2d, ai_tab, x2, zeros_tab)

    out = pl.pallas_call(
        _tc_combine_kernel,
        grid=grid,
        in_specs=[
            pl.BlockSpec((NC, BN, TW), lambda i: (0, i, 0)),
            pl.BlockSpec((BN, TW), lambda i: (i, 0)),
            pl.BlockSpec((BN, 16), lambda i: (i, 0)),
            pl.BlockSpec((BN, HD), lambda i: (i, 0)),
        ],
        out_specs=pl.BlockSpec((BN, HD), lambda i: (i, 0)),
        out_shape=jax.ShapeDtypeStruct((N, HD), f32),
    )(acc, x2, ai_tab, res)
    return out


# speedup vs baseline: 59.6906x; 59.6906x over previous
import jax
import jax.numpy as jnp
from jax import lax
from jax.experimental import pallas as pl
from jax.experimental.pallas import tpu as pltpu
from jax.experimental.pallas import tpu_sc as plsc

N = 10000
E = 320000
HEADS = 4
OUT_DIM = 32
HD = HEADS * OUT_DIM
TW = 144

NC = 2
NS = 16
NW = NC * NS
CHUNK = 128
CHUNKS_PER_TILE = 79
E_PAD = NW * CHUNKS_PER_TILE * CHUNK
ROWS_PER_TILE = N // NS
BN = 2000


def _tc_prep_kernel(x_ref, wl_ref, wr_ref, br_ref, bias_ref, wai_ref, waj_ref,
                    x2_ref, ai_ref, res_ref):
    x = x_ref[...]
    xl = jnp.dot(x, wl_ref[...].T, preferred_element_type=jnp.float32)
    x2_ref[:, 0:HD] = xl
    x2_ref[:, HD:TW] = jnp.dot(xl, waj_ref[...],
                               preferred_element_type=jnp.float32)
    ai_ref[...] = jnp.dot(xl, wai_ref[...], preferred_element_type=jnp.float32)
    res_ref[...] = (jnp.dot(x, wr_ref[...].T,
                            preferred_element_type=jnp.float32)
                    + br_ref[...] + bias_ref[...])


def _tc_combine_kernel(acc_ref, x2_ref, ai_ref, res_ref, out_ref):
    rows = lax.broadcasted_iota(jnp.int32, (16, HD), 0)
    cols = lax.broadcasted_iota(jnp.int32, (16, HD), 1)
    expand = (rows == cols // OUT_DIM).astype(jnp.float32)

    acc = acc_ref[0] + acc_ref[1]
    xl = x2_ref[:, 0:HD]
    s_self = ai_ref[...] + x2_ref[:, HD:TW]
    den_self = jnp.exp(jnp.maximum(s_self, 0.2 * s_self))
    num = acc[:, 0:HD] + jnp.dot(den_self, expand,
                                 preferred_element_type=jnp.float32) * xl
    den = jnp.dot(acc[:, HD:TW] + den_self, expand,
                  preferred_element_type=jnp.float32) + 1e-16
    out_ref[...] = num / den + res_ref[...]


def _bcast_lane(vec, lane):
    idx = jnp.full((16, 1), lane, dtype=jnp.int32)
    return lax.gather(
        vec, idx,
        lax.GatherDimensionNumbers(offset_dims=(), collapsed_slice_dims=(0,),
                                   start_index_map=(0,)),
        (1,), mode=lax.GatherScatterMode.PROMISE_IN_BOUNDS)


def _sc_edge_kernel(row_hbm, col_hbm, ai_hbm, x2_hbm, z_hbm, out_hbm,
                    ridx, cidx, aig, x2g, acc, sem_a, sem_b):
    cid = lax.axis_index("c")
    sid = lax.axis_index("s")
    wid = sid * NC + cid

    pltpu.sync_copy(z_hbm.at[pl.ds(sid * ROWS_PER_TILE, ROWS_PER_TILE)],
                    acc.at[pl.ds(sid * ROWS_PER_TILE, ROWS_PER_TILE)])
    plsc.subcore_barrier()

    chunk0 = wid * CHUNKS_PER_TILE

    @pl.loop(0, CHUNKS_PER_TILE)
    def _chunk(g):
        gi = chunk0 + g
        pltpu.sync_copy(row_hbm.at[pl.ds(gi, 1)], ridx)
        pltpu.sync_copy(col_hbm.at[pl.ds(gi, 1)], cidx)
        c1 = pltpu.async_copy(ai_hbm.at[ridx.at[0]], aig, sem_a)
        c2 = pltpu.async_copy(x2_hbm.at[cidx.at[0]], x2g, sem_b)
        c1.wait()
        c2.wait()

        @pl.loop(0, CHUNK // 16)
        def _grp(grp):
            ri = ridx[pl.ds(0, 1), pl.ds(grp * 16, 16)].reshape((16,))
            ci = cidx[pl.ds(0, 1), pl.ds(grp * 16, 16)].reshape((16,))
            vmask = jnp.where(ri != ci, 1.0, 0.0).astype(jnp.float32)
            for j in range(16):
                e = grp * 16 + j
                ai = aig[pl.ds(e, 1), :].reshape((16,))
                aj = x2g[pl.ds(e, 1), pl.ds(HD, 16)].reshape((16,))
                s = ai + aj
                s = jnp.maximum(s, 0.2 * s)
                esc = jnp.exp(s) * _bcast_lane(vmask, j)
                x2g[pl.ds(e, 1), pl.ds(HD, 16)] = esc.reshape((1, 16))
                for h in range(HEADS):
                    sc_h = _bcast_lane(esc, h)
                    for c in (2 * h, 2 * h + 1):
                        sl = (pl.ds(e, 1), pl.ds(c * 16, 16))
                        v = x2g[sl].reshape((16,))
                        x2g[sl] = (v * sc_h).reshape((1, 16))

        pltpu.sync_copy(x2g, acc.at[ridx.at[0]], add=True)

    plsc.subcore_barrier()
    pltpu.sync_copy(acc.at[pl.ds(sid * ROWS_PER_TILE, ROWS_PER_TILE)],
                    out_hbm.at[cid].at[pl.ds(sid * ROWS_PER_TILE,
                                             ROWS_PER_TILE)])


def kernel(x, W_lin, att_i, att_j, bias, W_res, b_res, edge_index):
    f32 = jnp.float32

    att_i_f = att_i.reshape(HEADS, OUT_DIM).astype(f32)
    att_j_f = att_j.reshape(HEADS, OUT_DIM).astype(f32)
    wai = jnp.zeros((HD, 16), f32)
    waj = jnp.zeros((HD, 16), f32)
    for h in range(HEADS):
        wai = wai.at[h * OUT_DIM:(h + 1) * OUT_DIM, h].set(att_i_f[h])
        waj = waj.at[h * OUT_DIM:(h + 1) * OUT_DIM, h].set(att_j_f[h])

    pad = jnp.zeros((E_PAD - E,), jnp.int32)
    row2d = jnp.concatenate([edge_index[0], pad]).reshape(-1, CHUNK)
    col2d = jnp.concatenate([edge_index[1], pad]).reshape(-1, CHUNK)
    zeros_tab = jnp.zeros((N, TW), f32)

    grid = (N // BN,)
    x2, ai_tab, res = pl.pallas_call(
        _tc_prep_kernel,
        grid=grid,
        in_specs=[
            pl.BlockSpec((BN, HD), lambda i: (i, 0)),
            pl.BlockSpec((HD, HD), lambda i: (0, 0)),
            pl.BlockSpec((HD, HD), lambda i: (0, 0)),
            pl.BlockSpec((HD,), lambda i: (0,)),
            pl.BlockSpec((HD,), lambda i: (0,)),
            pl.BlockSpec((HD, 16), lambda i: (0, 0)),
            pl.BlockSpec((HD, 16), lambda i: (0, 0)),
        ],
        out_specs=[
            pl.BlockSpec((BN, TW), lambda i: (i, 0)),
            pl.BlockSpec((BN, 16), lambda i: (i, 0)),
            pl.BlockSpec((BN, HD), lambda i: (i, 0)),
        ],
        out_shape=[
            jax.ShapeDtypeStruct((N, TW), f32),
            jax.ShapeDtypeStruct((N, 16), f32),
            jax.ShapeDtypeStruct((N, HD), f32),
        ],
    )(x, W_lin, W_res, b_res, bias, wai, waj)

    mesh = plsc.VectorSubcoreMesh(core_axis_name="c", subcore_axis_name="s")
    sc = pl.kernel(
        _sc_edge_kernel,
        out_type=jax.ShapeDtypeStruct((NC, N, TW), f32),
        mesh=mesh,
        compiler_params=pltpu.CompilerParams(use_tc_tiling_on_sc=False),
        scratch_types=[
            pltpu.VMEM((1, CHUNK), jnp.int32),
            pltpu.VMEM((1, CHUNK), jnp.int32),
            pltpu.VMEM((CHUNK, 16), f32),
            pltpu.VMEM((CHUNK, TW), f32),
            pltpu.VMEM_SHARED((N, TW), f32),
            pltpu.SemaphoreType.DMA,
            pltpu.SemaphoreType.DMA,
        ],
    )
    acc = sc(row2d, col2d, ai_tab, x2, zeros_tab)

    out = pl.pallas_call(
        _tc_combine_kernel,
        grid=grid,
        in_specs=[
            pl.BlockSpec((NC, BN, TW), lambda i: (0, i, 0)),
            pl.BlockSpec((BN, TW), lambda i: (i, 0)),
            pl.BlockSpec((BN, 16), lambda i: (i, 0)),
            pl.BlockSpec((BN, HD), lambda i: (i, 0)),
        ],
        out_specs=pl.BlockSpec((BN, HD), lambda i: (i, 0)),
        out_shape=jax.ShapeDtypeStruct((N, HD), f32),
    )(acc, x2, ai_tab, res)
    return out


# speedup vs baseline: 85.0518x; 1.4249x over previous
import jax
import jax.numpy as jnp
from jax import lax
from jax.experimental import pallas as pl
from jax.experimental.pallas import tpu as pltpu
from jax.experimental.pallas import tpu_sc as plsc

N = 10000
E = 320000
HEADS = 4
OUT_DIM = 32
HD = HEADS * OUT_DIM
TW = 144

NC = 2
NS = 16
NW = NC * NS
CHUNK = 112
CHUNKS_PER_TILE = 90
E_PAD = NW * CHUNKS_PER_TILE * CHUNK
ROWS_PER_TILE = N // NS
BN = 2000


def _tc_prep_kernel(x_ref, wl_ref, wr_ref, br_ref, bias_ref, wai_ref, waj_ref,
                    x2_ref, ai_ref, res_ref):
    x = x_ref[...]
    xl = jnp.dot(x, wl_ref[...].T, preferred_element_type=jnp.float32)
    x2_ref[:, 0:HD] = xl
    x2_ref[:, HD:TW] = jnp.dot(xl, waj_ref[...],
                               preferred_element_type=jnp.float32)
    ai_ref[...] = jnp.dot(xl, wai_ref[...], preferred_element_type=jnp.float32)
    res_ref[...] = (jnp.dot(x, wr_ref[...].T,
                            preferred_element_type=jnp.float32)
                    + br_ref[...] + bias_ref[...])


def _tc_combine_kernel(acc_ref, x2_ref, ai_ref, res_ref, out_ref):
    rows = lax.broadcasted_iota(jnp.int32, (16, HD), 0)
    cols = lax.broadcasted_iota(jnp.int32, (16, HD), 1)
    expand = (rows == cols // OUT_DIM).astype(jnp.float32)

    acc = acc_ref[0] + acc_ref[1]
    xl = x2_ref[:, 0:HD]
    s_self = ai_ref[...] + x2_ref[:, HD:TW]
    den_self = jnp.exp(jnp.maximum(s_self, 0.2 * s_self))
    num = acc[:, 0:HD] + jnp.dot(den_self, expand,
                                 preferred_element_type=jnp.float32) * xl
    den = jnp.dot(acc[:, HD:TW] + den_self, expand,
                  preferred_element_type=jnp.float32) + 1e-16
    out_ref[...] = num / den + res_ref[...]


def _bcast_lane(vec, lane):
    idx = jnp.full((16, 1), lane, dtype=jnp.int32)
    return lax.gather(
        vec, idx,
        lax.GatherDimensionNumbers(offset_dims=(), collapsed_slice_dims=(0,),
                                   start_index_map=(0,)),
        (1,), mode=lax.GatherScatterMode.PROMISE_IN_BOUNDS)


def _sc_edge_kernel(row_hbm, col_hbm, ai_hbm, x2_hbm, z_hbm, out_hbm,
                    ridx, cidx, aig, x2g, acc, isem, gsem):
    cid = lax.axis_index("c")
    sid = lax.axis_index("s")
    wid = sid * NC + cid
    chunk0 = wid * CHUNKS_PER_TILE
    NCH = CHUNKS_PER_TILE

    pltpu.sync_copy(z_hbm.at[pl.ds(sid * ROWS_PER_TILE, ROWS_PER_TILE)],
                    acc.at[pl.ds(sid * ROWS_PER_TILE, ROWS_PER_TILE)])
    plsc.subcore_barrier()

    def idx_fetch(b, gi):
        pltpu.async_copy(row_hbm.at[pl.ds(gi, 1)], ridx[b], isem[b])
        pltpu.async_copy(col_hbm.at[pl.ds(gi, 1)], cidx[b], isem[b])

    def idx_wait(b):
        pltpu.make_async_copy(row_hbm.at[pl.ds(0, 1)], ridx[b], isem[b]).wait()
        pltpu.make_async_copy(col_hbm.at[pl.ds(0, 1)], cidx[b], isem[b]).wait()

    def gather_issue(b):
        pltpu.async_copy(ai_hbm.at[ridx[b].at[0]], aig[b], gsem[b])
        pltpu.async_copy(x2_hbm.at[cidx[b].at[0]], x2g[b], gsem[b])

    def gather_wait(b):
        pltpu.make_async_copy(ai_hbm.at[ridx[b].at[0]], aig[b],
                              gsem[b]).wait()
        pltpu.make_async_copy(x2_hbm.at[cidx[b].at[0]], x2g[b],
                              gsem[b]).wait()

    def compute(b):
        @pl.loop(0, CHUNK // 16)
        def _grp(grp):
            ri = ridx[b][pl.ds(0, 1), pl.ds(grp * 16, 16)].reshape((16,))
            ci = cidx[b][pl.ds(0, 1), pl.ds(grp * 16, 16)].reshape((16,))
            vmask = jnp.where(ri != ci, 1.0, 0.0).astype(jnp.float32)
            for j in range(16):
                e = grp * 16 + j
                ai = aig[b][pl.ds(e, 1), :].reshape((16,))
                aj = x2g[b][pl.ds(e, 1), pl.ds(HD, 16)].reshape((16,))
                s = ai + aj
                s = jnp.maximum(s, 0.2 * s)
                esc = jnp.exp(s) * _bcast_lane(vmask, j)
                x2g[b][pl.ds(e, 1), pl.ds(HD, 16)] = esc.reshape((1, 16))
                for h in range(HEADS):
                    sc_h = _bcast_lane(esc, h)
                    for c in (2 * h, 2 * h + 1):
                        sl = (pl.ds(e, 1), pl.ds(c * 16, 16))
                        v = x2g[b][sl].reshape((16,))
                        x2g[b][sl] = (v * sc_h).reshape((1, 16))

    pltpu.sync_copy(row_hbm.at[pl.ds(chunk0, 1)], ridx[0])
    pltpu.sync_copy(col_hbm.at[pl.ds(chunk0, 1)], cidx[0])
    idx_fetch(1, chunk0 + 1)
    gather_issue(0)

    @pl.loop(0, NCH - 2, step=2)
    def _main(g0):
        for k in range(2):
            g = g0 + k
            b = k
            b1 = 1 - k
            gather_wait(b)
            idx_wait(b1)
            gather_issue(b1)
            compute(b)
            pltpu.sync_copy(x2g[b], acc.at[ridx[b].at[0]], add=True)
            idx_fetch(b, chunk0 + g + 2)

    gather_wait(0)
    idx_wait(1)
    gather_issue(1)
    compute(0)
    pltpu.sync_copy(x2g[0], acc.at[ridx[0].at[0]], add=True)

    gather_wait(1)
    compute(1)
    pltpu.sync_copy(x2g[1], acc.at[ridx[1].at[0]], add=True)

    plsc.subcore_barrier()
    pltpu.sync_copy(acc.at[pl.ds(sid * ROWS_PER_TILE, ROWS_PER_TILE)],
                    out_hbm.at[cid].at[pl.ds(sid * ROWS_PER_TILE,
                                             ROWS_PER_TILE)])


def kernel(x, W_lin, att_i, att_j, bias, W_res, b_res, edge_index):
    f32 = jnp.float32

    att_i_f = att_i.reshape(HEADS, OUT_DIM).astype(f32)
    att_j_f = att_j.reshape(HEADS, OUT_DIM).astype(f32)
    wai = jnp.zeros((HD, 16), f32)
    waj = jnp.zeros((HD, 16), f32)
    for h in range(HEADS):
        wai = wai.at[h * OUT_DIM:(h + 1) * OUT_DIM, h].set(att_i_f[h])
        waj = waj.at[h * OUT_DIM:(h + 1) * OUT_DIM, h].set(att_j_f[h])

    pad = jnp.zeros((E_PAD - E,), jnp.int32)
    row2d = jnp.concatenate([edge_index[0], pad]).reshape(-1, CHUNK)
    col2d = jnp.concatenate([edge_index[1], pad]).reshape(-1, CHUNK)
    zeros_tab = jnp.zeros((N, TW), f32)

    grid = (N // BN,)
    x2, ai_tab, res = pl.pallas_call(
        _tc_prep_kernel,
        grid=grid,
        in_specs=[
            pl.BlockSpec((BN, HD), lambda i: (i, 0)),
            pl.BlockSpec((HD, HD), lambda i: (0, 0)),
            pl.BlockSpec((HD, HD), lambda i: (0, 0)),
            pl.BlockSpec((HD,), lambda i: (0,)),
            pl.BlockSpec((HD,), lambda i: (0,)),
            pl.BlockSpec((HD, 16), lambda i: (0, 0)),
            pl.BlockSpec((HD, 16), lambda i: (0, 0)),
        ],
        out_specs=[
            pl.BlockSpec((BN, TW), lambda i: (i, 0)),
            pl.BlockSpec((BN, 16), lambda i: (i, 0)),
            pl.BlockSpec((BN, HD), lambda i: (i, 0)),
        ],
        out_shape=[
            jax.ShapeDtypeStruct((N, TW), f32),
            jax.ShapeDtypeStruct((N, 16), f32),
            jax.ShapeDtypeStruct((N, HD), f32),
        ],
    )(x, W_lin, W_res, b_res, bias, wai, waj)

    mesh = plsc.VectorSubcoreMesh(core_axis_name="c", subcore_axis_name="s")
    sc = pl.kernel(
        _sc_edge_kernel,
        out_type=jax.ShapeDtypeStruct((NC, N, TW), f32),
        mesh=mesh,
        compiler_params=pltpu.CompilerParams(use_tc_tiling_on_sc=False),
        scratch_types=[
            [pltpu.VMEM((1, CHUNK), jnp.int32) for _ in range(2)],
            [pltpu.VMEM((1, CHUNK), jnp.int32) for _ in range(2)],
            [pltpu.VMEM((CHUNK, 16), f32) for _ in range(2)],
            [pltpu.VMEM((CHUNK, TW), f32) for _ in range(2)],
            pltpu.VMEM_SHARED((N, TW), f32),
            [pltpu.SemaphoreType.DMA for _ in range(2)],
            [pltpu.SemaphoreType.DMA for _ in range(2)],
        ],
    )
    acc = sc(row2d, col2d, ai_tab, x2, zeros_tab)

    out = pl.pallas_call(
        _tc_combine_kernel,
        grid=grid,
        in_specs=[
            pl.BlockSpec((NC, BN, TW), lambda i: (0, i, 0)),
            pl.BlockSpec((BN, TW), lambda i: (i, 0)),
            pl.BlockSpec((BN, 16), lambda i: (i, 0)),
            pl.BlockSpec((BN, HD), lambda i: (i, 0)),
        ],
        out_specs=pl.BlockSpec((BN, HD), lambda i: (i, 0)),
        out_shape=jax.ShapeDtypeStruct((N, HD), f32),
    )(acc, x2, ai_tab, res)
    return out


# speedup vs baseline: 98.0962x; 1.1534x over previous
import jax
import jax.numpy as jnp
from jax import lax
from jax.experimental import pallas as pl
from jax.experimental.pallas import tpu as pltpu
from jax.experimental.pallas import tpu_sc as plsc

N = 10000
E = 320000
HEADS = 4
OUT_DIM = 32
HD = HEADS * OUT_DIM
TW = 144

NC = 2
NS = 16
NW = NC * NS
CHUNK = 80
CHUNKS_PER_TILE = 126
E_PAD = NW * CHUNKS_PER_TILE * CHUNK
ROWS_PER_TILE = N // NS
BN = 2000


def _tc_prep_kernel(x_ref, wl_ref, wr_ref, br_ref, bias_ref, wai_ref, waj_ref,
                    x2_ref, ai_ref, res_ref):
    x = x_ref[...]
    xl = jnp.dot(x, wl_ref[...].T, preferred_element_type=jnp.float32)
    x2_ref[:, 0:HD] = xl
    x2_ref[:, HD:TW] = jnp.dot(xl, waj_ref[...],
                               preferred_element_type=jnp.float32)
    ai_ref[...] = jnp.dot(xl, wai_ref[...], preferred_element_type=jnp.float32)
    res_ref[...] = (jnp.dot(x, wr_ref[...].T,
                            preferred_element_type=jnp.float32)
                    + br_ref[...] + bias_ref[...])


def _tc_combine_kernel(acc_ref, x2_ref, ai_ref, res_ref, out_ref):
    rows = lax.broadcasted_iota(jnp.int32, (16, HD), 0)
    cols = lax.broadcasted_iota(jnp.int32, (16, HD), 1)
    expand = (rows == cols // OUT_DIM).astype(jnp.float32)

    acc = acc_ref[0] + acc_ref[1]
    xl = x2_ref[:, 0:HD]
    s_self = ai_ref[...] + x2_ref[:, HD:TW]
    den_self = jnp.exp(jnp.maximum(s_self, 0.2 * s_self))
    num = acc[:, 0:HD] + jnp.dot(den_self, expand,
                                 preferred_element_type=jnp.float32) * xl
    den = jnp.dot(acc[:, HD:TW] + den_self, expand,
                  preferred_element_type=jnp.float32) + 1e-16
    out_ref[...] = num / den + res_ref[...]


def _bcast_lane(vec, lane):
    idx = jnp.full((16, 1), lane, dtype=jnp.int32)
    return lax.gather(
        vec, idx,
        lax.GatherDimensionNumbers(offset_dims=(), collapsed_slice_dims=(0,),
                                   start_index_map=(0,)),
        (1,), mode=lax.GatherScatterMode.PROMISE_IN_BOUNDS)


def _sc_edge_kernel(row_hbm, col_hbm, ai_hbm, x2_hbm, z_hbm, out_hbm,
                    ridx, cidx, sridx, scidx, aig, x2g, acc,
                    isem, gsem, ssem):
    cid = lax.axis_index("c")
    sid = lax.axis_index("s")
    wid = sid * NC + cid
    chunk0 = wid * CHUNKS_PER_TILE
    NCH = CHUNKS_PER_TILE

    pltpu.sync_copy(z_hbm.at[pl.ds(sid * ROWS_PER_TILE, ROWS_PER_TILE)],
                    acc.at[pl.ds(sid * ROWS_PER_TILE, ROWS_PER_TILE)])
    plsc.subcore_barrier()

    def idx_fetch(b, gi):
        pltpu.async_copy(row_hbm.at[pl.ds(gi, 1)], ridx[b], isem[b])
        pltpu.async_copy(col_hbm.at[pl.ds(gi, 1)], cidx[b], isem[b])

    def idx_wait(b):
        pltpu.make_async_copy(row_hbm.at[pl.ds(0, 1)], ridx[b], isem[b]).wait()
        pltpu.make_async_copy(col_hbm.at[pl.ds(0, 1)], cidx[b], isem[b]).wait()

    def gather_issue(b):
        pltpu.async_copy(ai_hbm.at[ridx[b].at[0]], aig[b], gsem[b])
        pltpu.async_copy(x2_hbm.at[cidx[b].at[0]], x2g[b], gsem[b])

    def gather_wait(b):
        pltpu.make_async_copy(ai_hbm.at[ridx[b].at[0]], aig[b],
                              gsem[b]).wait()
        pltpu.make_async_copy(x2_hbm.at[cidx[b].at[0]], x2g[b],
                              gsem[b]).wait()

    def scatter_issue(b):
        pltpu.async_copy(x2g[b], acc.at[sridx[b].at[0]], ssem[b], add=True)

    def scatter_wait(b):
        pltpu.make_async_copy(x2g[b], acc.at[sridx[b].at[0]], ssem[b]).wait()

    def snapshot_idx(b):
        for t in range(CHUNK // 16):
            sl = (pl.ds(0, 1), pl.ds(t * 16, 16))
            sridx[b][sl] = ridx[b][sl]
            scidx[b][sl] = cidx[b][sl]

    def compute(b):
        @pl.loop(0, CHUNK // 16)
        def _grp(grp):
            ri = sridx[b][pl.ds(0, 1), pl.ds(grp * 16, 16)].reshape((16,))
            ci = scidx[b][pl.ds(0, 1), pl.ds(grp * 16, 16)].reshape((16,))
            vmask = jnp.where(ri != ci, 1.0, 0.0).astype(jnp.float32)
            for j in range(16):
                e = grp * 16 + j
                ai = aig[b][pl.ds(e, 1), :].reshape((16,))
                aj = x2g[b][pl.ds(e, 1), pl.ds(HD, 16)].reshape((16,))
                s = ai + aj
                s = jnp.maximum(s, 0.2 * s)
                esc = jnp.exp(s) * _bcast_lane(vmask, j)
                x2g[b][pl.ds(e, 1), pl.ds(HD, 16)] = esc.reshape((1, 16))
                for h in range(HEADS):
                    sc_h = _bcast_lane(esc, h)
                    for c in (2 * h, 2 * h + 1):
                        sl = (pl.ds(e, 1), pl.ds(c * 16, 16))
                        v = x2g[b][sl].reshape((16,))
                        x2g[b][sl] = (v * sc_h).reshape((1, 16))

    pltpu.sync_copy(row_hbm.at[pl.ds(chunk0, 1)], ridx[0])
    pltpu.sync_copy(col_hbm.at[pl.ds(chunk0, 1)], cidx[0])
    idx_fetch(1, chunk0 + 1)
    idx_fetch(2, chunk0 + 2)
    gather_issue(0)

    @pl.loop(0, NCH - 3, step=3)
    def _main(g0):
        for k in range(3):
            g = g0 + k
            b = k
            b1 = (k + 1) % 3
            gather_wait(b)
            snapshot_idx(b)
            idx_fetch(b, chunk0 + g + 3)

            @pl.when(g >= 2)
            def _():
                scatter_wait(b1)

            idx_wait(b1)
            gather_issue(b1)
            compute(b)
            scatter_issue(b)

    gather_wait(0)
    snapshot_idx(0)
    scatter_wait(1)
    idx_wait(1)
    gather_issue(1)
    compute(0)
    scatter_issue(0)

    gather_wait(1)
    snapshot_idx(1)
    scatter_wait(2)
    idx_wait(2)
    gather_issue(2)
    compute(1)
    scatter_issue(1)

    gather_wait(2)
    snapshot_idx(2)
    scatter_wait(0)
    compute(2)
    scatter_issue(2)

    scatter_wait(1)
    scatter_wait(2)

    plsc.subcore_barrier()
    pltpu.sync_copy(acc.at[pl.ds(sid * ROWS_PER_TILE, ROWS_PER_TILE)],
                    out_hbm.at[cid].at[pl.ds(sid * ROWS_PER_TILE,
                                             ROWS_PER_TILE)])


def kernel(x, W_lin, att_i, att_j, bias, W_res, b_res, edge_index):
    f32 = jnp.float32

    att_i_f = att_i.reshape(HEADS, OUT_DIM).astype(f32)
    att_j_f = att_j.reshape(HEADS, OUT_DIM).astype(f32)
    wai = jnp.zeros((HD, 16), f32)
    waj = jnp.zeros((HD, 16), f32)
    for h in range(HEADS):
        wai = wai.at[h * OUT_DIM:(h + 1) * OUT_DIM, h].set(att_i_f[h])
        waj = waj.at[h * OUT_DIM:(h + 1) * OUT_DIM, h].set(att_j_f[h])

    pad = jnp.zeros((E_PAD - E,), jnp.int32)
    row2d = jnp.concatenate([edge_index[0], pad]).reshape(-1, CHUNK)
    col2d = jnp.concatenate([edge_index[1], pad]).reshape(-1, CHUNK)
    zeros_tab = jnp.zeros((N, TW), f32)

    grid = (N // BN,)
    x2, ai_tab, res = pl.pallas_call(
        _tc_prep_kernel,
        grid=grid,
        in_specs=[
            pl.BlockSpec((BN, HD), lambda i: (i, 0)),
            pl.BlockSpec((HD, HD), lambda i: (0, 0)),
            pl.BlockSpec((HD, HD), lambda i: (0, 0)),
            pl.BlockSpec((HD,), lambda i: (0,)),
            pl.BlockSpec((HD,), lambda i: (0,)),
            pl.BlockSpec((HD, 16), lambda i: (0, 0)),
            pl.BlockSpec((HD, 16), lambda i: (0, 0)),
        ],
        out_specs=[
            pl.BlockSpec((BN, TW), lambda i: (i, 0)),
            pl.BlockSpec((BN, 16), lambda i: (i, 0)),
            pl.BlockSpec((BN, HD), lambda i: (i, 0)),
        ],
        out_shape=[
            jax.ShapeDtypeStruct((N, TW), f32),
            jax.ShapeDtypeStruct((N, 16), f32),
            jax.ShapeDtypeStruct((N, HD), f32),
        ],
    )(x, W_lin, W_res, b_res, bias, wai, waj)

    mesh = plsc.VectorSubcoreMesh(core_axis_name="c", subcore_axis_name="s")
    sc = pl.kernel(
        _sc_edge_kernel,
        out_type=jax.ShapeDtypeStruct((NC, N, TW), f32),
        mesh=mesh,
        compiler_params=pltpu.CompilerParams(use_tc_tiling_on_sc=False),
        scratch_types=[
            [pltpu.VMEM((1, CHUNK), jnp.int32) for _ in range(3)],
            [pltpu.VMEM((1, CHUNK), jnp.int32) for _ in range(3)],
            [pltpu.VMEM((1, CHUNK), jnp.int32) for _ in range(3)],
            [pltpu.VMEM((1, CHUNK), jnp.int32) for _ in range(3)],
            [pltpu.VMEM((CHUNK, 16), f32) for _ in range(3)],
            [pltpu.VMEM((CHUNK, TW), f32) for _ in range(3)],
            pltpu.VMEM_SHARED((N, TW), f32),
            [pltpu.SemaphoreType.DMA for _ in range(3)],
            [pltpu.SemaphoreType.DMA for _ in range(3)],
            [pltpu.SemaphoreType.DMA for _ in range(3)],
        ],
    )
    acc = sc(row2d, col2d, ai_tab, x2, zeros_tab)

    out = pl.pallas_call(
        _tc_combine_kernel,
        grid=grid,
        in_specs=[
            pl.BlockSpec((NC, BN, TW), lambda i: (0, i, 0)),
            pl.BlockSpec((BN, TW), lambda i: (i, 0)),
            pl.BlockSpec((BN, 16), lambda i: (i, 0)),
            pl.BlockSpec((BN, HD), lambda i: (i, 0)),
        ],
        out_specs=pl.BlockSpec((BN, HD), lambda i: (i, 0)),
        out_shape=jax.ShapeDtypeStruct((N, HD), f32),
    )(acc, x2, ai_tab, res)
    return out
